# baseline (speedup 1.0000x reference)
import functools

import numpy as np

import jax
import jax.numpy as jnp
from jax import lax
from jax.experimental import pallas as pl
from jax.experimental.pallas import tpu as pltpu
from jax.experimental.pallas import tpu_sc as plsc

NUM_CORES = 2
NUM_SUBCORES = 16
NUM_WORKERS = NUM_CORES * NUM_SUBCORES
CHUNK = 128
EMB = 128
LANES = 16


def _sc_gather_add(attr_table, obj_table, va, vo, p_pad):
    rows_per_w = p_pad // NUM_WORKERS
    n_chunks = rows_per_w // CHUNK
    assert n_chunks % 2 == 0
    mesh = plsc.VectorSubcoreMesh(core_axis_name="c", subcore_axis_name="s")
    va3 = va.reshape(NUM_WORKERS, n_chunks, CHUNK)
    vo3 = vo.reshape(NUM_WORKERS, n_chunks, CHUNK)

    n_rows = attr_table.shape[0]

    @functools.partial(
        pl.kernel,
        mesh=mesh,
        out_type=jax.ShapeDtypeStruct((p_pad, EMB // 2), jnp.int32),
        scratch_types=[
            pltpu.VMEM_SHARED((n_rows, EMB), jnp.float32),
            pltpu.VMEM_SHARED((n_rows, EMB), jnp.float32),
            pltpu.VMEM((n_chunks, CHUNK), jnp.int32),
            pltpu.VMEM((n_chunks, CHUNK), jnp.int32),
            pltpu.VMEM((CHUNK, EMB), jnp.float32),
            pltpu.VMEM((CHUNK, EMB), jnp.float32),
            pltpu.VMEM((CHUNK, EMB), jnp.float32),
            pltpu.VMEM((CHUNK, EMB), jnp.float32),
            pltpu.VMEM((CHUNK, EMB // 2), jnp.int32),
            pltpu.VMEM((CHUNK, EMB // 2), jnp.int32),
            pltpu.SemaphoreType.DMA,
            pltpu.SemaphoreType.DMA,
            pltpu.SemaphoreType.DMA,
            pltpu.SemaphoreType.DMA,
            pltpu.SemaphoreType.DMA,
            pltpu.SemaphoreType.DMA,
        ],
    )
    def body(attr_hbm, obj_hbm, va_hbm, vo_hbm, out_hbm,
             attr_s, obj_s, ia_v, io_v, ra0, rb0, ra1, rb1, rc0, rc1,
             sa0, sb0, sa1, sb1, so0, so1):
        wid = lax.axis_index("s") * NUM_CORES + lax.axis_index("c")
        base = wid * rows_per_w

        @pl.when(lax.axis_index("s") == 0)
        def _():
            pltpu.sync_copy(attr_hbm, attr_s)
            pltpu.sync_copy(obj_hbm, obj_s)

        plsc.subcore_barrier()
        pltpu.sync_copy(va_hbm.at[wid], ia_v)
        pltpu.sync_copy(vo_hbm.at[wid], io_v)

        def fire(c, ra, rb, sa, sb):
            pltpu.async_copy(attr_s.at[ia_v.at[c]], ra, sa)
            pltpu.async_copy(obj_s.at[io_v.at[c]], rb, sb)

        def out_slice(c):
            return out_hbm.at[pl.ds(base + c * CHUNK, CHUNK)]

        def drain_add_store_refire(c_cur, c_next, ra, rb, rc, sa, sb, so):
            pltpu.make_async_copy(attr_s.at[ia_v.at[c_cur]], ra, sa).wait()
            pltpu.make_async_copy(obj_s.at[io_v.at[c_cur]], rb, sb).wait()

            @pl.when(c_cur >= 2)
            def _():
                pltpu.make_async_copy(rc, out_slice(c_cur), so).wait()

            def row_step(r, c2):
                rnd = jnp.full((LANES,), 0x8000, dtype=jnp.int32)
                msk = jnp.full((LANES,), -65536, dtype=jnp.int32)
                for j2 in range(EMB // (2 * LANES)):
                    lo = ra[r, pl.ds(2 * j2 * LANES, LANES)] + rb[r, pl.ds(2 * j2 * LANES, LANES)]
                    hi = ra[r, pl.ds((2 * j2 + 1) * LANES, LANES)] + rb[r, pl.ds((2 * j2 + 1) * LANES, LANES)]
                    lo_b = lax.bitcast_convert_type(lo, jnp.int32) + rnd
                    hi_b = lax.bitcast_convert_type(hi, jnp.int32) + rnd
                    word = (lax.shift_right_logical(lo_b, 16)
                            | (hi_b & msk))
                    rc[r, pl.ds(j2 * LANES, LANES)] = word
                return c2

            lax.fori_loop(0, CHUNK, row_step, 0)
            pltpu.async_copy(rc, out_slice(c_cur), so)

            @pl.when(c_next < n_chunks)
            def _():
                fire(c_next, ra, rb, sa, sb)

        fire(0, ra0, rb0, sa0, sb0)
        fire(1, ra1, rb1, sa1, sb1)

        def step(i, carry):
            drain_add_store_refire(2 * i, 2 * i + 2, ra0, rb0, rc0, sa0, sb0, so0)
            drain_add_store_refire(2 * i + 1, 2 * i + 3, ra1, rb1, rc1, sa1, sb1, so1)
            return carry

        lax.fori_loop(0, n_chunks // 2, step, 0)
        pltpu.make_async_copy(rc0, out_slice(n_chunks - 2), so0).wait()
        pltpu.make_async_copy(rc1, out_slice(n_chunks - 1), so1).wait()

    return body(attr_table, obj_table, va3, vo3)


def _unpack_pair(w):
    lo = lax.bitcast_convert_type(lax.shift_left(w, 16), jnp.float32)
    hi = lax.bitcast_convert_type(
        jnp.bitwise_and(w, jnp.int32(-65536)), jnp.float32)
    return jnp.concatenate([lo, hi], axis=1)


def _tc_scores_chunk(x, pair_k, prev_out, n_pairs, col_base, tile_p):
    batch = x.shape[0]
    chunk_rows = pair_k.shape[0]
    real_cols = min(col_base + chunk_rows, n_pairs) - col_base
    grid = (real_cols + tile_p - 1) // tile_p
    base_blk = col_base // tile_p

    def body(x_ref, p_ref, _, o_ref):
        xv = x_ref[...]
        xn = xv * (1.0 / (jnp.sqrt(jnp.sum(xv * xv, axis=1, keepdims=True)) + 1e-8))
        pv = _unpack_pair(p_ref[...])
        pinv = 1.0 / (jnp.sqrt(jnp.sum(pv * pv, axis=1, keepdims=True)) + 1e-8)
        pn = pv * pinv
        o_ref[...] = lax.dot_general(
            xn, pn, (((1,), (1,)), ((), ())),
            preferred_element_type=jnp.float32)

    return pl.pallas_call(
        body,
        grid=(grid,),
        in_specs=[
            pl.BlockSpec((batch, EMB), lambda j: (0, 0)),
            pl.BlockSpec((tile_p, EMB // 2), lambda j: (j, 0)),
            pl.BlockSpec(memory_space=pl.ANY),
        ],
        out_specs=pl.BlockSpec((batch, tile_p), lambda j: (0, base_blk + j)),
        out_shape=jax.ShapeDtypeStruct((batch, n_pairs), jnp.float32),
        input_output_aliases={2: 0},
    )(x, pair_k, prev_out)


def kernel(x, val_attrs, val_objs, attr_table, obj_table):
    n_pairs = val_attrs.shape[0]
    quantum = 2 * NUM_WORKERS * CHUNK
    chunk_quanta = [13]
    n_quanta = sum(chunk_quanta)
    assert n_quanta * quantum >= n_pairs
    p_pad = n_quanta * quantum
    pad_idx = jnp.arange(p_pad - n_pairs, dtype=jnp.int32) % attr_table.shape[0]
    va = jnp.concatenate([val_attrs.astype(jnp.int32), pad_idx])
    vo = jnp.concatenate([val_objs.astype(jnp.int32), pad_idx])

    k = np.arange(EMB // 2, dtype=np.int32)
    perm = np.concatenate([32 * (k // 16) + k % 16,
                           32 * (k // 16) + 16 + k % 16])
    xp = jnp.take(x, jnp.asarray(perm), axis=1)

    pairs, bases = [], []
    col_base = 0
    for q in chunk_quanta:
        rows = q * quantum
        pairs.append(_sc_gather_add(
            attr_table, obj_table,
            lax.dynamic_slice_in_dim(va, col_base, rows),
            lax.dynamic_slice_in_dim(vo, col_base, rows),
            rows))
        bases.append(col_base)
        col_base += rows

    out = _tc_scores_first(xp, pairs[0], n_pairs, tile_p=4096)
    for pair_k, base in zip(pairs[1:], bases[1:]):
        out = _tc_scores_chunk(xp, pair_k, out, n_pairs, base, tile_p=4096)
    return out


def _tc_scores_first(x, pair_k, n_pairs, tile_p):
    batch = x.shape[0]
    chunk_rows = pair_k.shape[0]
    grid = (min(chunk_rows, n_pairs) + tile_p - 1) // tile_p

    def body(x_ref, p_ref, o_ref):
        xv = x_ref[...]
        xn = xv * (1.0 / (jnp.sqrt(jnp.sum(xv * xv, axis=1, keepdims=True)) + 1e-8))
        pv = _unpack_pair(p_ref[...])
        pinv = 1.0 / (jnp.sqrt(jnp.sum(pv * pv, axis=1, keepdims=True)) + 1e-8)
        pn = pv * pinv
        o_ref[...] = lax.dot_general(
            xn, pn, (((1,), (1,)), ((), ())),
            preferred_element_type=jnp.float32)

    return pl.pallas_call(
        body,
        grid=(grid,),
        in_specs=[
            pl.BlockSpec((batch, EMB), lambda j: (0, 0)),
            pl.BlockSpec((tile_p, EMB // 2), lambda j: (j, 0)),
        ],
        out_specs=pl.BlockSpec((batch, tile_p), lambda j: (0, j)),
        out_shape=jax.ShapeDtypeStruct((batch, n_pairs), jnp.float32),
    )(x, pair_k)


# speedup vs baseline: 1.0162x; 1.0162x over previous
import functools

import numpy as np

import jax
import jax.numpy as jnp
from jax import lax
from jax.experimental import pallas as pl
from jax.experimental.pallas import tpu as pltpu
from jax.experimental.pallas import tpu_sc as plsc

NUM_CORES = 2
NUM_SUBCORES = 16
NUM_WORKERS = NUM_CORES * NUM_SUBCORES
CHUNK = 128
EMB = 128
LANES = 16


def _sc_gather_add(attr_table, obj_table, va, vo, p_pad):
    rows_per_w = p_pad // NUM_WORKERS
    n_chunks = rows_per_w // CHUNK
    assert n_chunks % 2 == 0
    mesh = plsc.VectorSubcoreMesh(core_axis_name="c", subcore_axis_name="s")
    va3 = va.reshape(NUM_WORKERS, n_chunks, CHUNK)
    vo3 = vo.reshape(NUM_WORKERS, n_chunks, CHUNK)

    n_rows = attr_table.shape[0]

    @functools.partial(
        pl.kernel,
        mesh=mesh,
        out_type=jax.ShapeDtypeStruct((p_pad, EMB // 2), jnp.int32),
        scratch_types=[
            pltpu.VMEM_SHARED((n_rows, EMB), jnp.float32),
            pltpu.VMEM_SHARED((n_rows, EMB), jnp.float32),
            pltpu.VMEM((n_chunks, CHUNK), jnp.int32),
            pltpu.VMEM((n_chunks, CHUNK), jnp.int32),
            pltpu.VMEM((CHUNK, EMB), jnp.float32),
            pltpu.VMEM((CHUNK, EMB), jnp.float32),
            pltpu.VMEM((CHUNK, EMB), jnp.float32),
            pltpu.VMEM((CHUNK, EMB), jnp.float32),
            pltpu.VMEM((CHUNK, EMB // 2), jnp.int32),
            pltpu.VMEM((CHUNK, EMB // 2), jnp.int32),
            pltpu.SemaphoreType.DMA,
            pltpu.SemaphoreType.DMA,
            pltpu.SemaphoreType.DMA,
            pltpu.SemaphoreType.DMA,
            pltpu.SemaphoreType.DMA,
            pltpu.SemaphoreType.DMA,
        ],
    )
    def body(attr_hbm, obj_hbm, va_hbm, vo_hbm, out_hbm,
             attr_s, obj_s, ia_v, io_v, ra0, rb0, ra1, rb1, rc0, rc1,
             sa0, sb0, sa1, sb1, so0, so1):
        wid = lax.axis_index("s") * NUM_CORES + lax.axis_index("c")
        base = wid * rows_per_w

        @pl.when(lax.axis_index("s") == 0)
        def _():
            pltpu.sync_copy(attr_hbm, attr_s)
            pltpu.sync_copy(obj_hbm, obj_s)

        plsc.subcore_barrier()
        pltpu.sync_copy(va_hbm.at[wid], ia_v)
        pltpu.sync_copy(vo_hbm.at[wid], io_v)

        def fire(c, ra, rb, sa, sb):
            pltpu.async_copy(attr_s.at[ia_v.at[c]], ra, sa)
            pltpu.async_copy(obj_s.at[io_v.at[c]], rb, sb)

        def out_slice(c):
            return out_hbm.at[pl.ds(base + c * CHUNK, CHUNK)]

        def drain_add_store_refire(c_cur, c_next, ra, rb, rc, sa, sb, so):
            pltpu.make_async_copy(attr_s.at[ia_v.at[c_cur]], ra, sa).wait()
            pltpu.make_async_copy(obj_s.at[io_v.at[c_cur]], rb, sb).wait()

            @pl.when(c_cur >= 2)
            def _():
                pltpu.make_async_copy(rc, out_slice(c_cur), so).wait()

            def row_step(r, c2):
                rnd = jnp.full((LANES,), 0x8000, dtype=jnp.int32)
                msk = jnp.full((LANES,), -65536, dtype=jnp.int32)
                for j2 in range(EMB // (2 * LANES)):
                    lo = ra[r, pl.ds(2 * j2 * LANES, LANES)] + rb[r, pl.ds(2 * j2 * LANES, LANES)]
                    hi = ra[r, pl.ds((2 * j2 + 1) * LANES, LANES)] + rb[r, pl.ds((2 * j2 + 1) * LANES, LANES)]
                    lo_b = lax.bitcast_convert_type(lo, jnp.int32) + rnd
                    hi_b = lax.bitcast_convert_type(hi, jnp.int32) + rnd
                    word = (lax.shift_right_logical(lo_b, 16)
                            | (hi_b & msk))
                    rc[r, pl.ds(j2 * LANES, LANES)] = word
                return c2

            lax.fori_loop(0, CHUNK, row_step, 0)
            pltpu.async_copy(rc, out_slice(c_cur), so)

            @pl.when(c_next < n_chunks)
            def _():
                fire(c_next, ra, rb, sa, sb)

        fire(0, ra0, rb0, sa0, sb0)
        fire(1, ra1, rb1, sa1, sb1)

        def step(i, carry):
            drain_add_store_refire(2 * i, 2 * i + 2, ra0, rb0, rc0, sa0, sb0, so0)
            drain_add_store_refire(2 * i + 1, 2 * i + 3, ra1, rb1, rc1, sa1, sb1, so1)
            return carry

        lax.fori_loop(0, n_chunks // 2, step, 0)
        pltpu.make_async_copy(rc0, out_slice(n_chunks - 2), so0).wait()
        pltpu.make_async_copy(rc1, out_slice(n_chunks - 1), so1).wait()

    return body(attr_table, obj_table, va3, vo3)


def _unpack_pair(w):
    lo = lax.bitcast_convert_type(lax.shift_left(w, 16), jnp.float32)
    hi = lax.bitcast_convert_type(
        jnp.bitwise_and(w, jnp.int32(-65536)), jnp.float32)
    return jnp.concatenate([lo, hi], axis=1)


def _tc_scores_chunk(x, pair_k, prev_out, n_pairs, col_base, tile_p):
    batch = x.shape[0]
    chunk_rows = pair_k.shape[0]
    real_cols = min(col_base + chunk_rows, n_pairs) - col_base
    grid = (real_cols + tile_p - 1) // tile_p
    base_blk = col_base // tile_p

    def body(x_ref, p_ref, _, o_ref):
        xv = x_ref[...]
        xn = xv * (1.0 / (jnp.sqrt(jnp.sum(xv * xv, axis=1, keepdims=True)) + 1e-8))
        pv = _unpack_pair(p_ref[...])
        pinv = 1.0 / (jnp.sqrt(jnp.sum(pv * pv, axis=1, keepdims=True)) + 1e-8)
        pn = pv * pinv
        o_ref[...] = lax.dot_general(
            xn, pn, (((1,), (1,)), ((), ())),
            preferred_element_type=jnp.float32)

    return pl.pallas_call(
        body,
        grid=(grid,),
        in_specs=[
            pl.BlockSpec((batch, EMB), lambda j: (0, 0)),
            pl.BlockSpec((tile_p, EMB // 2), lambda j: (j, 0)),
            pl.BlockSpec(memory_space=pl.ANY),
        ],
        out_specs=pl.BlockSpec((batch, tile_p), lambda j: (0, base_blk + j)),
        out_shape=jax.ShapeDtypeStruct((batch, n_pairs), jnp.float32),
        input_output_aliases={2: 0},
    )(x, pair_k, prev_out)


def kernel(x, val_attrs, val_objs, attr_table, obj_table):
    n_pairs = val_attrs.shape[0]
    quantum = 2 * NUM_WORKERS * CHUNK
    chunk_quanta = [1, 4, 4, 4]
    n_quanta = sum(chunk_quanta)
    assert n_quanta * quantum >= n_pairs
    p_pad = n_quanta * quantum
    pad_idx = jnp.arange(p_pad - n_pairs, dtype=jnp.int32) % attr_table.shape[0]
    va = jnp.concatenate([val_attrs.astype(jnp.int32), pad_idx])
    vo = jnp.concatenate([val_objs.astype(jnp.int32), pad_idx])

    k = np.arange(EMB // 2, dtype=np.int32)
    perm = np.concatenate([32 * (k // 16) + k % 16,
                           32 * (k // 16) + 16 + k % 16])
    xp = jnp.take(x, jnp.asarray(perm), axis=1)

    pairs, bases = [], []
    col_base = 0
    for q in chunk_quanta:
        rows = q * quantum
        pairs.append(_sc_gather_add(
            attr_table, obj_table,
            lax.dynamic_slice_in_dim(va, col_base, rows),
            lax.dynamic_slice_in_dim(vo, col_base, rows),
            rows))
        bases.append(col_base)
        col_base += rows

    out = _tc_scores_first(xp, pairs[0], n_pairs, tile_p=4096)
    for pair_k, base in zip(pairs[1:], bases[1:]):
        out = _tc_scores_chunk(xp, pair_k, out, n_pairs, base, tile_p=4096)
    return out


def _tc_scores_first(x, pair_k, n_pairs, tile_p):
    batch = x.shape[0]
    chunk_rows = pair_k.shape[0]
    grid = (min(chunk_rows, n_pairs) + tile_p - 1) // tile_p

    def body(x_ref, p_ref, o_ref):
        xv = x_ref[...]
        xn = xv * (1.0 / (jnp.sqrt(jnp.sum(xv * xv, axis=1, keepdims=True)) + 1e-8))
        pv = _unpack_pair(p_ref[...])
        pinv = 1.0 / (jnp.sqrt(jnp.sum(pv * pv, axis=1, keepdims=True)) + 1e-8)
        pn = pv * pinv
        o_ref[...] = lax.dot_general(
            xn, pn, (((1,), (1,)), ((), ())),
            preferred_element_type=jnp.float32)

    return pl.pallas_call(
        body,
        grid=(grid,),
        in_specs=[
            pl.BlockSpec((batch, EMB), lambda j: (0, 0)),
            pl.BlockSpec((tile_p, EMB // 2), lambda j: (j, 0)),
        ],
        out_specs=pl.BlockSpec((batch, tile_p), lambda j: (0, j)),
        out_shape=jax.ShapeDtypeStruct((batch, n_pairs), jnp.float32),
    )(x, pair_k)
